# Initial kernel scaffold; baseline (speedup 1.0000x reference)
import functools

import jax
import jax.numpy as jnp
from jax import lax
from jax.experimental import pallas as pl
from jax.experimental.pallas import tpu as pltpu
from jax.experimental.pallas import tpu_sc as plsc

N = 10000
D = 128
G = 128
NCLS = 10
NP = 10240
E = 320000
NCORES = 2
NSUB = 16
NTILES = NCORES * NSUB
LW = 128
CH = 80
EP = NTILES * CH * LW
RPT = NP // NSUB
BLK = 1000
NBLK = 10
EPS = 1e-5

_mesh = plsc.VectorSubcoreMesh(core_axis_name="c", subcore_axis_name="s")


def _sc_degree(row_d):
    @functools.partial(
        pl.kernel,
        out_type=jax.ShapeDtypeStruct((NCORES, NP, 16), jnp.float32),
        mesh=_mesh,
        scratch_types=[
            pltpu.VMEM((CH, LW), jnp.int32),
            pltpu.VMEM((LW, 16), jnp.float32),
            pltpu.VMEM((LW, 16), jnp.float32),
            pltpu.VMEM_SHARED((NP, 16), jnp.float32),
        ],
    )
    def k(row_hbm, out_hbm, row_v, ones_v, zero_v, acc):
        c = lax.axis_index("c")
        s = lax.axis_index("s")
        wid = c * NSUB + s
        pltpu.sync_copy(row_hbm.at[wid], row_v)

        def fill(r, carry):
            ones_v[r, pl.ds(0, 16)] = jnp.ones((16,), jnp.float32)
            zero_v[r, pl.ds(0, 16)] = jnp.zeros((16,), jnp.float32)
            return carry

        lax.fori_loop(0, LW, fill, 0)
        for t in range(RPT // LW):
            pltpu.sync_copy(zero_v, acc.at[pl.ds(s * RPT + t * LW, LW)])
        plsc.subcore_barrier()

        def body(j, carry):
            pltpu.sync_copy(ones_v, acc.at[row_v.at[j]], add=True)
            return carry

        lax.fori_loop(0, CH, body, 0)
        plsc.subcore_barrier()
        pltpu.sync_copy(acc.at[pl.ds(s * RPT, RPT)],
                        out_hbm.at[c, pl.ds(s * RPT, RPT)])

    return k(row_d)


def _sc_propagate(hs, row_g, col_s):
    @functools.partial(
        pl.kernel,
        out_type=jax.ShapeDtypeStruct((NCORES, NP, D), jnp.float32),
        mesh=_mesh,
        scratch_types=[
            pltpu.VMEM((CH, LW), jnp.int32),
            pltpu.VMEM((CH, LW), jnp.int32),
            pltpu.VMEM((LW, D), jnp.float32),
            pltpu.VMEM((LW, D), jnp.float32),
            pltpu.VMEM_SHARED((NP, D), jnp.float32),
            pltpu.SemaphoreType.DMA,
        ],
    )
    def k(hs_hbm, row_hbm, col_hbm, out_hbm, row_v, col_v, gbuf, zbuf, acc, sem):
        c = lax.axis_index("c")
        s = lax.axis_index("s")
        wid = c * NSUB + s
        pltpu.sync_copy(row_hbm.at[wid], row_v)
        pltpu.sync_copy(col_hbm.at[wid], col_v)

        def zrow(r, carry):
            for j in range(D // 16):
                zbuf[r, pl.ds(16 * j, 16)] = jnp.zeros((16,), jnp.float32)
            return carry

        lax.fori_loop(0, LW, zrow, 0)
        for t in range(RPT // LW):
            pltpu.sync_copy(zbuf, acc.at[pl.ds(s * RPT + t * LW, LW)])
        plsc.subcore_barrier()

        def body(j, carry):
            pltpu.async_copy(hs_hbm.at[row_v.at[j]], gbuf, sem).wait()
            pltpu.sync_copy(gbuf, acc.at[col_v.at[j]], add=True)
            return carry

        lax.fori_loop(0, CH, body, 0)
        plsc.subcore_barrier()
        pltpu.sync_copy(acc.at[pl.ds(s * RPT, RPT)],
                        out_hbm.at[c, pl.ds(s * RPT, RPT)])

    return k(hs, row_g, col_s)


def _stats(arr):
    def body(x_ref, o_ref, acc):
        i = pl.program_id(0)

        @pl.when(i == 0)
        def _():
            acc[...] = jnp.zeros_like(acc)

        xb = x_ref[...]
        acc[...] += jnp.concatenate(
            [jnp.sum(xb, 0, keepdims=True), jnp.sum(xb * xb, 0, keepdims=True)], 0)
        o_ref[...] = acc[...]

    return pl.pallas_call(
        body,
        grid=(NBLK,),
        in_specs=[pl.BlockSpec((BLK, D), lambda i: (i, 0))],
        out_specs=pl.BlockSpec((2, D), lambda i: (0, 0)),
        out_shape=jax.ShapeDtypeStruct((2, D), jnp.float32),
        scratch_shapes=[pltpu.VMEM((2, D), jnp.float32)],
    )(arr)


def _bn_coeff(s_ref, g_ref, b_ref):
    s = s_ref[...]
    m = s[0:1, :] / N
    v = s[1:2, :] / N - m * m
    a = g_ref[...] * lax.rsqrt(v + EPS)
    return a, b_ref[...] - a * m


def _layer0(x, W_feat, stats_x, g0, b0):
    def body(x_ref, w_ref, s_ref, g_ref, b_ref, h_ref, st_ref, acc):
        i = pl.program_id(0)
        a, cc = _bn_coeff(s_ref, g_ref, b_ref)
        h = jnp.maximum(
            jnp.dot(x_ref[...] * a + cc, w_ref[...],
                    preferred_element_type=jnp.float32), 0.0)
        h_ref[...] = h

        @pl.when(i == 0)
        def _():
            acc[...] = jnp.zeros_like(acc)

        acc[...] += jnp.concatenate(
            [jnp.sum(h, 0, keepdims=True), jnp.sum(h * h, 0, keepdims=True)], 0)
        st_ref[...] = acc[...]

    return pl.pallas_call(
        body,
        grid=(NBLK,),
        in_specs=[
            pl.BlockSpec((BLK, D), lambda i: (i, 0)),
            pl.BlockSpec((D, D), lambda i: (0, 0)),
            pl.BlockSpec((2, D), lambda i: (0, 0)),
            pl.BlockSpec((1, D), lambda i: (0, 0)),
            pl.BlockSpec((1, D), lambda i: (0, 0)),
        ],
        out_specs=[
            pl.BlockSpec((BLK, D), lambda i: (i, 0)),
            pl.BlockSpec((2, D), lambda i: (0, 0)),
        ],
        out_shape=[
            jax.ShapeDtypeStruct((N, D), jnp.float32),
            jax.ShapeDtypeStruct((2, D), jnp.float32),
        ],
        scratch_shapes=[pltpu.VMEM((2, D), jnp.float32)],
    )(x, W_feat, stats_x, g0, b0)


def _dis(d_ref):
    cnt = d_ref[0, :, 0:1] + d_ref[1, :, 0:1] + 1.0
    return lax.rsqrt(cnt)


def _kt(h, W, stats, g_row, b_row, degp):
    def body(h_ref, w_ref, s_ref, g_ref, b_ref, d_ref, hs_ref):
        a, cc = _bn_coeff(s_ref, g_ref, b_ref)
        hb = h_ref[...] * a + cc
        hs_ref[...] = _dis(d_ref) * jnp.dot(
            hb, w_ref[...], preferred_element_type=jnp.float32)

    return pl.pallas_call(
        body,
        grid=(NBLK,),
        in_specs=[
            pl.BlockSpec((BLK, D), lambda i: (i, 0)),
            pl.BlockSpec((D, D), lambda i: (0, 0)),
            pl.BlockSpec((2, D), lambda i: (0, 0)),
            pl.BlockSpec((1, D), lambda i: (0, 0)),
            pl.BlockSpec((1, D), lambda i: (0, 0)),
            pl.BlockSpec((2, BLK, 16), lambda i: (0, i, 0)),
        ],
        out_specs=pl.BlockSpec((BLK, D), lambda i: (i, 0)),
        out_shape=jax.ShapeDtypeStruct((N, D), jnp.float32),
    )(h, W, stats, g_row, b_row, degp)


def _ku(p, hs, degp, b_row):
    def body(p_ref, hs_ref, d_ref, b_ref, h_ref, st_ref, acc):
        i = pl.program_id(0)
        hn = jnp.maximum(
            _dis(d_ref) * (p_ref[0] + p_ref[1] + hs_ref[...]) + b_ref[...], 0.0)
        h_ref[...] = hn

        @pl.when(i == 0)
        def _():
            acc[...] = jnp.zeros_like(acc)

        acc[...] += jnp.concatenate(
            [jnp.sum(hn, 0, keepdims=True), jnp.sum(hn * hn, 0, keepdims=True)], 0)
        st_ref[...] = acc[...]

    return pl.pallas_call(
        body,
        grid=(NBLK,),
        in_specs=[
            pl.BlockSpec((2, BLK, D), lambda i: (0, i, 0)),
            pl.BlockSpec((BLK, D), lambda i: (i, 0)),
            pl.BlockSpec((2, BLK, 16), lambda i: (0, i, 0)),
            pl.BlockSpec((1, D), lambda i: (0, 0)),
        ],
        out_specs=[
            pl.BlockSpec((BLK, D), lambda i: (i, 0)),
            pl.BlockSpec((2, D), lambda i: (0, 0)),
        ],
        out_shape=[
            jax.ShapeDtypeStruct((N, D), jnp.float32),
            jax.ShapeDtypeStruct((2, D), jnp.float32),
        ],
        scratch_shapes=[pltpu.VMEM((2, D), jnp.float32)],
    )(p, hs, degp, b_row)


def _ku_pool(p, hs, degp, b_row, batch3):
    def body(p_ref, hs_ref, d_ref, b_ref, seg_ref, g_ref, acc):
        i = pl.program_id(0)
        hn = jnp.maximum(
            _dis(d_ref) * (p_ref[0] + p_ref[1] + hs_ref[...]) + b_ref[...], 0.0)
        seg = seg_ref[0]
        oh = (lax.broadcasted_iota(jnp.int32, (G, BLK), 0) == seg
              ).astype(jnp.float32)

        @pl.when(i == 0)
        def _():
            acc[...] = jnp.zeros_like(acc)

        acc[...] += jnp.dot(oh, hn, preferred_element_type=jnp.float32)
        g_ref[...] = acc[...]

    return pl.pallas_call(
        body,
        grid=(NBLK,),
        in_specs=[
            pl.BlockSpec((2, BLK, D), lambda i: (0, i, 0)),
            pl.BlockSpec((BLK, D), lambda i: (i, 0)),
            pl.BlockSpec((2, BLK, 16), lambda i: (0, i, 0)),
            pl.BlockSpec((1, D), lambda i: (0, 0)),
            pl.BlockSpec((1, 1, BLK), lambda i: (i, 0, 0)),
        ],
        out_specs=pl.BlockSpec((G, D), lambda i: (0, 0)),
        out_shape=jax.ShapeDtypeStruct((G, D), jnp.float32),
        scratch_shapes=[pltpu.VMEM((G, D), jnp.float32)],
    )(p, hs, degp, b_row, batch3)


def _head(g, g4, b4, g5, b5, W_fc, bfc, W_cls, bcls):
    def body(g_ref, g4_ref, b4_ref, g5_ref, b5_ref, wfc_ref, bfc_ref,
             wcls_ref, bcls_ref, o_ref):
        gg = g_ref[...]
        m = jnp.mean(gg, 0, keepdims=True)
        v = jnp.mean(gg * gg, 0, keepdims=True) - m * m
        h = g4_ref[...] * lax.rsqrt(v + EPS) * (gg - m) + b4_ref[...]
        h = jnp.maximum(
            jnp.dot(h, wfc_ref[...], preferred_element_type=jnp.float32)
            + bfc_ref[...], 0.0)
        m2 = jnp.mean(h, 0, keepdims=True)
        v2 = jnp.mean(h * h, 0, keepdims=True) - m2 * m2
        h = g5_ref[...] * lax.rsqrt(v2 + EPS) * (h - m2) + b5_ref[...]
        lo = (jnp.dot(h, wcls_ref[...], preferred_element_type=jnp.float32)
              + bcls_ref[...])
        mx = jnp.max(lo, -1, keepdims=True)
        ls = mx + jnp.log(jnp.sum(jnp.exp(lo - mx), -1, keepdims=True))
        o_ref[...] = lo - ls

    return pl.pallas_call(
        body,
        out_shape=jax.ShapeDtypeStruct((G, NCLS), jnp.float32),
    )(g, g4, b4, g5, b5, W_fc, bfc, W_cls, bcls)


def kernel(x, edge_index, batch, W_feat, W1, b1, W2, b2, W3, b3,
           W_fc, b_fc, W_cls, b_cls, bn_g, bn_b):
    row = edge_index[0]
    col = edge_index[1]
    pad = EP - E
    eids = jnp.arange(pad, dtype=jnp.int32)
    row_g = jnp.concatenate([row, eids % 16]).reshape(NTILES, CH, LW)
    row_d = jnp.concatenate([row, N + (eids % 16)]).reshape(NTILES, CH, LW)
    col_s = jnp.concatenate([col, N + (eids % 16)]).reshape(NTILES, CH, LW)

    degp = _sc_degree(row_d)
    stats_x = _stats(x)
    h, st = _layer0(x, W_feat, stats_x, bn_g[0:1], bn_b[0:1])
    g = None
    for li, (W, b) in enumerate(((W1, b1), (W2, b2), (W3, b3))):
        hs = _kt(h, W, st, bn_g[1 + li:2 + li], bn_b[1 + li:2 + li], degp)
        p = _sc_propagate(hs, row_g, col_s)
        if li < 2:
            h, st = _ku(p, hs, degp, b.reshape(1, D))
        else:
            g = _ku_pool(p, hs, degp, b.reshape(1, D),
                         batch.reshape(NBLK, 1, BLK))
    return _head(g, bn_g[4:5], bn_b[4:5], bn_g[5:6], bn_b[5:6],
                 W_fc, b_fc.reshape(1, D), W_cls, b_cls.reshape(1, NCLS))


# speedup vs baseline: 12.8871x; 12.8871x over previous
import functools

import jax
import jax.numpy as jnp
from jax import lax
from jax.experimental import pallas as pl
from jax.experimental.pallas import tpu as pltpu
from jax.experimental.pallas import tpu_sc as plsc

N = 10000
D = 128
HD = 64
G = 128
NCLS = 10
NP = 10240
E = 320000
NCORES = 2
NSUB = 16
NTILES = NCORES * NSUB
LW = 128
CH = 80
CH2 = 160
EP = NTILES * CH * LW
RPT = NP // NSUB
BLK = 1000
NBLK = 10
EPS = 1e-5

_mesh = plsc.VectorSubcoreMesh(core_axis_name="c", subcore_axis_name="s")


def _sc_degree(row_d):
    @functools.partial(
        pl.kernel,
        out_type=jax.ShapeDtypeStruct((NCORES, NP, 16), jnp.float32),
        mesh=_mesh,
        scratch_types=[
            pltpu.VMEM((CH, LW), jnp.int32),
            pltpu.VMEM((LW, 16), jnp.float32),
            pltpu.VMEM((LW, 16), jnp.float32),
            pltpu.VMEM_SHARED((NP, 16), jnp.float32),
        ],
    )
    def k(row_hbm, out_hbm, row_v, ones_v, zero_v, acc):
        c = lax.axis_index("c")
        s = lax.axis_index("s")
        wid = c * NSUB + s
        pltpu.sync_copy(row_hbm.at[wid], row_v)

        def fill(r, carry):
            ones_v[r, pl.ds(0, 16)] = jnp.ones((16,), jnp.float32)
            zero_v[r, pl.ds(0, 16)] = jnp.zeros((16,), jnp.float32)
            return carry

        lax.fori_loop(0, LW, fill, 0)
        for t in range(RPT // LW):
            pltpu.sync_copy(zero_v, acc.at[pl.ds(s * RPT + t * LW, LW)])
        plsc.subcore_barrier()

        def body(j, carry):
            pltpu.sync_copy(ones_v, acc.at[row_v.at[j]], add=True)
            return carry

        lax.fori_loop(0, CH, body, 0)
        plsc.subcore_barrier()
        pltpu.sync_copy(acc.at[pl.ds(s * RPT, RPT)],
                        out_hbm.at[c, pl.ds(s * RPT, RPT)])

    return k(row_d)


def _sc_propagate(hsplit, row_g, col_s):
    @functools.partial(
        pl.kernel,
        out_type=jax.ShapeDtypeStruct((NCORES, NP, HD), jnp.float32),
        mesh=_mesh,
        scratch_types=[
            pltpu.VMEM((CH2, LW), jnp.int32),
            pltpu.VMEM((CH2, LW), jnp.int32),
            pltpu.VMEM((LW, HD), jnp.float32),
            pltpu.VMEM((LW, HD), jnp.float32),
            pltpu.VMEM_SHARED((NP, HD), jnp.float32),
            pltpu.SemaphoreType.DMA,
        ],
        compiler_params=pltpu.CompilerParams(use_tc_tiling_on_sc=False),
    )
    def k(hs_hbm, row_hbm, col_hbm, out_hbm, row_v, col_v, gbuf, zbuf, acc, sem):
        c = lax.axis_index("c")
        s = lax.axis_index("s")
        pltpu.sync_copy(row_hbm.at[s], row_v)
        pltpu.sync_copy(col_hbm.at[s], col_v)

        def zrow(r, carry):
            for j in range(HD // 16):
                zbuf[r, pl.ds(16 * j, 16)] = jnp.zeros((16,), jnp.float32)
            return carry

        lax.fori_loop(0, LW, zrow, 0)
        for t in range(RPT // LW):
            pltpu.sync_copy(zbuf, acc.at[pl.ds(s * RPT + t * LW, LW)])
        plsc.subcore_barrier()

        def body(j, carry):
            pltpu.async_copy(hs_hbm.at[c].at[row_v.at[j]], gbuf, sem).wait()
            pltpu.sync_copy(gbuf, acc.at[col_v.at[j]], add=True)
            return carry

        lax.fori_loop(0, CH2, body, 0)
        plsc.subcore_barrier()
        pltpu.sync_copy(acc.at[pl.ds(s * RPT, RPT)],
                        out_hbm.at[c, pl.ds(s * RPT, RPT)])

    return k(hsplit, row_g, col_s)


def _stats(arr):
    def body(x_ref, o_ref, acc):
        i = pl.program_id(0)

        @pl.when(i == 0)
        def _():
            acc[...] = jnp.zeros_like(acc)

        xb = x_ref[...]
        acc[...] += jnp.concatenate(
            [jnp.sum(xb, 0, keepdims=True), jnp.sum(xb * xb, 0, keepdims=True)], 0)
        o_ref[...] = acc[...]

    return pl.pallas_call(
        body,
        grid=(NBLK,),
        in_specs=[pl.BlockSpec((BLK, D), lambda i: (i, 0))],
        out_specs=pl.BlockSpec((2, D), lambda i: (0, 0)),
        out_shape=jax.ShapeDtypeStruct((2, D), jnp.float32),
        scratch_shapes=[pltpu.VMEM((2, D), jnp.float32)],
    )(arr)


def _bn_coeff(s_ref, g_ref, b_ref):
    s = s_ref[...]
    m = s[0:1, :] / N
    v = s[1:2, :] / N - m * m
    a = g_ref[...] * lax.rsqrt(v + EPS)
    return a, b_ref[...] - a * m


def _layer0(x, W_feat, stats_x, g0, b0):
    def body(x_ref, w_ref, s_ref, g_ref, b_ref, h_ref, st_ref, acc):
        i = pl.program_id(0)
        a, cc = _bn_coeff(s_ref, g_ref, b_ref)
        h = jnp.maximum(
            jnp.dot(x_ref[...] * a + cc, w_ref[...],
                    preferred_element_type=jnp.float32), 0.0)
        h_ref[...] = h

        @pl.when(i == 0)
        def _():
            acc[...] = jnp.zeros_like(acc)

        acc[...] += jnp.concatenate(
            [jnp.sum(h, 0, keepdims=True), jnp.sum(h * h, 0, keepdims=True)], 0)
        st_ref[...] = acc[...]

    return pl.pallas_call(
        body,
        grid=(NBLK,),
        in_specs=[
            pl.BlockSpec((BLK, D), lambda i: (i, 0)),
            pl.BlockSpec((D, D), lambda i: (0, 0)),
            pl.BlockSpec((2, D), lambda i: (0, 0)),
            pl.BlockSpec((1, D), lambda i: (0, 0)),
            pl.BlockSpec((1, D), lambda i: (0, 0)),
        ],
        out_specs=[
            pl.BlockSpec((BLK, D), lambda i: (i, 0)),
            pl.BlockSpec((2, D), lambda i: (0, 0)),
        ],
        out_shape=[
            jax.ShapeDtypeStruct((N, D), jnp.float32),
            jax.ShapeDtypeStruct((2, D), jnp.float32),
        ],
        scratch_shapes=[pltpu.VMEM((2, D), jnp.float32)],
    )(x, W_feat, stats_x, g0, b0)


def _dis(d_ref):
    cnt = d_ref[0, :, 0:1] + d_ref[1, :, 0:1] + 1.0
    return lax.rsqrt(cnt)


def _kt(h, W, stats, g_row, b_row, degp):
    def body(h_ref, w_ref, s_ref, g_ref, b_ref, d_ref, hs_ref):
        a, cc = _bn_coeff(s_ref, g_ref, b_ref)
        hb = h_ref[...] * a + cc
        hs = _dis(d_ref) * jnp.dot(
            hb, w_ref[...], preferred_element_type=jnp.float32)
        hs_ref[...] = jnp.stack([hs[:, :HD], hs[:, HD:]], axis=0)

    return pl.pallas_call(
        body,
        grid=(NBLK,),
        in_specs=[
            pl.BlockSpec((BLK, D), lambda i: (i, 0)),
            pl.BlockSpec((D, D), lambda i: (0, 0)),
            pl.BlockSpec((2, D), lambda i: (0, 0)),
            pl.BlockSpec((1, D), lambda i: (0, 0)),
            pl.BlockSpec((1, D), lambda i: (0, 0)),
            pl.BlockSpec((2, BLK, 16), lambda i: (0, i, 0)),
        ],
        out_specs=pl.BlockSpec((2, BLK, HD), lambda i: (0, i, 0)),
        out_shape=jax.ShapeDtypeStruct((2, N, HD), jnp.float32),
    )(h, W, stats, g_row, b_row, degp)


def _hs_full(hs_ref):
    return jnp.concatenate([hs_ref[0], hs_ref[1]], axis=1)


def _ku(p, hsplit, degp, b_row):
    def body(p_ref, hs_ref, d_ref, b_ref, h_ref, st_ref, acc):
        i = pl.program_id(0)
        hn = jnp.maximum(
            _dis(d_ref) * (_hs_full(p_ref) + _hs_full(hs_ref)) + b_ref[...], 0.0)
        h_ref[...] = hn

        @pl.when(i == 0)
        def _():
            acc[...] = jnp.zeros_like(acc)

        acc[...] += jnp.concatenate(
            [jnp.sum(hn, 0, keepdims=True), jnp.sum(hn * hn, 0, keepdims=True)], 0)
        st_ref[...] = acc[...]

    return pl.pallas_call(
        body,
        grid=(NBLK,),
        in_specs=[
            pl.BlockSpec((2, BLK, HD), lambda i: (0, i, 0)),
            pl.BlockSpec((2, BLK, HD), lambda i: (0, i, 0)),
            pl.BlockSpec((2, BLK, 16), lambda i: (0, i, 0)),
            pl.BlockSpec((1, D), lambda i: (0, 0)),
        ],
        out_specs=[
            pl.BlockSpec((BLK, D), lambda i: (i, 0)),
            pl.BlockSpec((2, D), lambda i: (0, 0)),
        ],
        out_shape=[
            jax.ShapeDtypeStruct((N, D), jnp.float32),
            jax.ShapeDtypeStruct((2, D), jnp.float32),
        ],
        scratch_shapes=[pltpu.VMEM((2, D), jnp.float32)],
    )(p, hsplit, degp, b_row)


def _ku_pool(p, hsplit, degp, b_row, batch3):
    def body(p_ref, hs_ref, d_ref, b_ref, seg_ref, g_ref, acc):
        i = pl.program_id(0)
        hn = jnp.maximum(
            _dis(d_ref) * (_hs_full(p_ref) + _hs_full(hs_ref)) + b_ref[...], 0.0)
        seg = seg_ref[0]
        oh = (lax.broadcasted_iota(jnp.int32, (G, BLK), 0) == seg
              ).astype(jnp.float32)

        @pl.when(i == 0)
        def _():
            acc[...] = jnp.zeros_like(acc)

        acc[...] += jnp.dot(oh, hn, preferred_element_type=jnp.float32)
        g_ref[...] = acc[...]

    return pl.pallas_call(
        body,
        grid=(NBLK,),
        in_specs=[
            pl.BlockSpec((2, BLK, HD), lambda i: (0, i, 0)),
            pl.BlockSpec((2, BLK, HD), lambda i: (0, i, 0)),
            pl.BlockSpec((2, BLK, 16), lambda i: (0, i, 0)),
            pl.BlockSpec((1, D), lambda i: (0, 0)),
            pl.BlockSpec((1, 1, BLK), lambda i: (i, 0, 0)),
        ],
        out_specs=pl.BlockSpec((G, D), lambda i: (0, 0)),
        out_shape=jax.ShapeDtypeStruct((G, D), jnp.float32),
        scratch_shapes=[pltpu.VMEM((G, D), jnp.float32)],
    )(p, hsplit, degp, b_row, batch3)


def _head(g, g4, b4, g5, b5, W_fc, bfc, W_cls, bcls):
    def body(g_ref, g4_ref, b4_ref, g5_ref, b5_ref, wfc_ref, bfc_ref,
             wcls_ref, bcls_ref, o_ref):
        gg = g_ref[...]
        m = jnp.mean(gg, 0, keepdims=True)
        v = jnp.mean(gg * gg, 0, keepdims=True) - m * m
        h = g4_ref[...] * lax.rsqrt(v + EPS) * (gg - m) + b4_ref[...]
        h = jnp.maximum(
            jnp.dot(h, wfc_ref[...], preferred_element_type=jnp.float32)
            + bfc_ref[...], 0.0)
        m2 = jnp.mean(h, 0, keepdims=True)
        v2 = jnp.mean(h * h, 0, keepdims=True) - m2 * m2
        h = g5_ref[...] * lax.rsqrt(v2 + EPS) * (h - m2) + b5_ref[...]
        lo = (jnp.dot(h, wcls_ref[...], preferred_element_type=jnp.float32)
              + bcls_ref[...])
        mx = jnp.max(lo, -1, keepdims=True)
        ls = mx + jnp.log(jnp.sum(jnp.exp(lo - mx), -1, keepdims=True))
        o_ref[...] = lo - ls

    return pl.pallas_call(
        body,
        out_shape=jax.ShapeDtypeStruct((G, NCLS), jnp.float32),
    )(g, g4, b4, g5, b5, W_fc, bfc, W_cls, bcls)


def kernel(x, edge_index, batch, W_feat, W1, b1, W2, b2, W3, b3,
           W_fc, b_fc, W_cls, b_cls, bn_g, bn_b):
    row = edge_index[0]
    col = edge_index[1]
    pad = EP - E
    eids = jnp.arange(pad, dtype=jnp.int32)
    row_g = jnp.concatenate([row, eids % 16]).reshape(NSUB, CH2, LW)
    col_s = jnp.concatenate([col, N + (eids % 16)]).reshape(NSUB, CH2, LW)
    row_d = jnp.concatenate([row, N + (eids % 16)]).reshape(NTILES, CH, LW)

    degp = _sc_degree(row_d)
    stats_x = _stats(x)
    h, st = _layer0(x, W_feat, stats_x, bn_g[0:1], bn_b[0:1])
    g = None
    for li, (W, b) in enumerate(((W1, b1), (W2, b2), (W3, b3))):
        hsplit = _kt(h, W, st, bn_g[1 + li:2 + li], bn_b[1 + li:2 + li], degp)
        p = _sc_propagate(hsplit, row_g, col_s)
        if li < 2:
            h, st = _ku(p, hsplit, degp, b.reshape(1, D))
        else:
            g = _ku_pool(p, hsplit, degp, b.reshape(1, D),
                         batch.reshape(NBLK, 1, BLK))
    return _head(g, bn_g[4:5], bn_b[4:5], bn_g[5:6], bn_b[5:6],
                 W_fc, b_fc.reshape(1, D), W_cls, b_cls.reshape(1, NCLS))


# speedup vs baseline: 15.7669x; 1.2235x over previous
import functools

import jax
import jax.numpy as jnp
from jax import lax
from jax.experimental import pallas as pl
from jax.experimental.pallas import tpu as pltpu
from jax.experimental.pallas import tpu_sc as plsc

N = 10000
D = 128
HD = 64
G = 128
NCLS = 10
NP = 10240
E = 320000
NCORES = 2
NSUB = 16
NTILES = NCORES * NSUB
LW = 128
CH = 80
CH2 = 160
EP = NTILES * CH * LW
RPT = NP // NSUB
BLK = 1000
NBLK = 10
EPS = 1e-5

_mesh = plsc.VectorSubcoreMesh(core_axis_name="c", subcore_axis_name="s")


def _sc_degree(row_d):
    @functools.partial(
        pl.kernel,
        out_type=jax.ShapeDtypeStruct((NCORES, NP, 16), jnp.float32),
        mesh=_mesh,
        scratch_types=[
            pltpu.VMEM((CH, LW), jnp.int32),
            pltpu.VMEM((LW, 16), jnp.float32),
            pltpu.VMEM((LW, 16), jnp.float32),
            pltpu.VMEM_SHARED((NP, 16), jnp.float32),
        ],
    )
    def k(row_hbm, out_hbm, row_v, ones_v, zero_v, acc):
        c = lax.axis_index("c")
        s = lax.axis_index("s")
        wid = c * NSUB + s
        pltpu.sync_copy(row_hbm.at[wid], row_v)

        def fill(r, carry):
            ones_v[r, pl.ds(0, 16)] = jnp.ones((16,), jnp.float32)
            zero_v[r, pl.ds(0, 16)] = jnp.zeros((16,), jnp.float32)
            return carry

        lax.fori_loop(0, LW, fill, 0)
        for t in range(RPT // LW):
            pltpu.sync_copy(zero_v, acc.at[pl.ds(s * RPT + t * LW, LW)])
        plsc.subcore_barrier()

        def body(j, carry):
            pltpu.sync_copy(ones_v, acc.at[row_v.at[j]], add=True)
            return carry

        lax.fori_loop(0, CH, body, 0)
        plsc.subcore_barrier()
        pltpu.sync_copy(acc.at[pl.ds(s * RPT, RPT)],
                        out_hbm.at[c, pl.ds(s * RPT, RPT)])

    return k(row_d)


def _sc_propagate(hsplit, row_g, col_s):
    @functools.partial(
        pl.kernel,
        out_type=jax.ShapeDtypeStruct((NCORES, NP, HD), jnp.float32),
        mesh=_mesh,
        scratch_types=[
            pltpu.VMEM((CH2, LW), jnp.int32),
            pltpu.VMEM((CH2, LW), jnp.int32),
            pltpu.VMEM((LW, HD), jnp.float32),
            pltpu.VMEM((LW, HD), jnp.float32),
            pltpu.VMEM((LW, HD), jnp.float32),
            pltpu.VMEM_SHARED((NP, HD), jnp.float32),
            pltpu.SemaphoreType.DMA,
            pltpu.SemaphoreType.DMA,
        ],
        compiler_params=pltpu.CompilerParams(use_tc_tiling_on_sc=False),
    )
    def k(hs_hbm, row_hbm, col_hbm, out_hbm, row_v, col_v, gbuf0, gbuf1, zbuf,
          acc, sem0, sem1):
        c = lax.axis_index("c")
        s = lax.axis_index("s")
        pltpu.sync_copy(row_hbm.at[s], row_v)
        pltpu.sync_copy(col_hbm.at[s], col_v)

        def zrow(r, carry):
            for j in range(HD // 16):
                zbuf[r, pl.ds(16 * j, 16)] = jnp.zeros((16,), jnp.float32)
            return carry

        lax.fori_loop(0, LW, zrow, 0)
        for t in range(RPT // LW):
            pltpu.sync_copy(zbuf, acc.at[pl.ds(s * RPT + t * LW, LW)])
        plsc.subcore_barrier()

        pltpu.async_copy(hs_hbm.at[c].at[row_v.at[0]], gbuf0, sem0)

        def body(i, carry):
            j0 = 2 * i
            for b, (gb, sem, gb_n, sem_n) in enumerate(
                    ((gbuf0, sem0, gbuf1, sem1), (gbuf1, sem1, gbuf0, sem0))):
                j = j0 + b
                pltpu.make_async_copy(
                    hs_hbm.at[c].at[row_v.at[j]], gb, sem).wait()

                @pl.when(j + 1 < CH2)
                def _():
                    pltpu.async_copy(
                        hs_hbm.at[c].at[row_v.at[j + 1]], gb_n, sem_n)

                pltpu.sync_copy(gb, acc.at[col_v.at[j]], add=True)
            return carry

        lax.fori_loop(0, CH2 // 2, body, 0)
        plsc.subcore_barrier()
        pltpu.sync_copy(acc.at[pl.ds(s * RPT, RPT)],
                        out_hbm.at[c, pl.ds(s * RPT, RPT)])

    return k(hsplit, row_g, col_s)


def _stats(arr):
    def body(x_ref, o_ref, acc):
        i = pl.program_id(0)

        @pl.when(i == 0)
        def _():
            acc[...] = jnp.zeros_like(acc)

        xb = x_ref[...]
        acc[...] += jnp.concatenate(
            [jnp.sum(xb, 0, keepdims=True), jnp.sum(xb * xb, 0, keepdims=True)], 0)
        o_ref[...] = acc[...]

    return pl.pallas_call(
        body,
        grid=(NBLK,),
        in_specs=[pl.BlockSpec((BLK, D), lambda i: (i, 0))],
        out_specs=pl.BlockSpec((2, D), lambda i: (0, 0)),
        out_shape=jax.ShapeDtypeStruct((2, D), jnp.float32),
        scratch_shapes=[pltpu.VMEM((2, D), jnp.float32)],
    )(arr)


def _bn_coeff(s_ref, g_ref, b_ref):
    s = s_ref[...]
    m = s[0:1, :] / N
    v = s[1:2, :] / N - m * m
    a = g_ref[...] * lax.rsqrt(v + EPS)
    return a, b_ref[...] - a * m


def _layer0(x, W_feat, stats_x, g0, b0):
    def body(x_ref, w_ref, s_ref, g_ref, b_ref, h_ref, st_ref, acc):
        i = pl.program_id(0)
        a, cc = _bn_coeff(s_ref, g_ref, b_ref)
        h = jnp.maximum(
            jnp.dot(x_ref[...] * a + cc, w_ref[...],
                    preferred_element_type=jnp.float32), 0.0)
        h_ref[...] = h

        @pl.when(i == 0)
        def _():
            acc[...] = jnp.zeros_like(acc)

        acc[...] += jnp.concatenate(
            [jnp.sum(h, 0, keepdims=True), jnp.sum(h * h, 0, keepdims=True)], 0)
        st_ref[...] = acc[...]

    return pl.pallas_call(
        body,
        grid=(NBLK,),
        in_specs=[
            pl.BlockSpec((BLK, D), lambda i: (i, 0)),
            pl.BlockSpec((D, D), lambda i: (0, 0)),
            pl.BlockSpec((2, D), lambda i: (0, 0)),
            pl.BlockSpec((1, D), lambda i: (0, 0)),
            pl.BlockSpec((1, D), lambda i: (0, 0)),
        ],
        out_specs=[
            pl.BlockSpec((BLK, D), lambda i: (i, 0)),
            pl.BlockSpec((2, D), lambda i: (0, 0)),
        ],
        out_shape=[
            jax.ShapeDtypeStruct((N, D), jnp.float32),
            jax.ShapeDtypeStruct((2, D), jnp.float32),
        ],
        scratch_shapes=[pltpu.VMEM((2, D), jnp.float32)],
    )(x, W_feat, stats_x, g0, b0)


def _dis(d_ref):
    cnt = d_ref[0, :, 0:1] + d_ref[1, :, 0:1] + 1.0
    return lax.rsqrt(cnt)


def _kt(h, W, stats, g_row, b_row, degp):
    def body(h_ref, w_ref, s_ref, g_ref, b_ref, d_ref, hs_ref):
        a, cc = _bn_coeff(s_ref, g_ref, b_ref)
        hb = h_ref[...] * a + cc
        hs = _dis(d_ref) * jnp.dot(
            hb, w_ref[...], preferred_element_type=jnp.float32)
        hs_ref[...] = jnp.stack([hs[:, :HD], hs[:, HD:]], axis=0)

    return pl.pallas_call(
        body,
        grid=(NBLK,),
        in_specs=[
            pl.BlockSpec((BLK, D), lambda i: (i, 0)),
            pl.BlockSpec((D, D), lambda i: (0, 0)),
            pl.BlockSpec((2, D), lambda i: (0, 0)),
            pl.BlockSpec((1, D), lambda i: (0, 0)),
            pl.BlockSpec((1, D), lambda i: (0, 0)),
            pl.BlockSpec((2, BLK, 16), lambda i: (0, i, 0)),
        ],
        out_specs=pl.BlockSpec((2, BLK, HD), lambda i: (0, i, 0)),
        out_shape=jax.ShapeDtypeStruct((2, N, HD), jnp.float32),
    )(h, W, stats, g_row, b_row, degp)


def _hs_full(hs_ref):
    return jnp.concatenate([hs_ref[0], hs_ref[1]], axis=1)


def _ku(p, hsplit, degp, b_row):
    def body(p_ref, hs_ref, d_ref, b_ref, h_ref, st_ref, acc):
        i = pl.program_id(0)
        hn = jnp.maximum(
            _dis(d_ref) * (_hs_full(p_ref) + _hs_full(hs_ref)) + b_ref[...], 0.0)
        h_ref[...] = hn

        @pl.when(i == 0)
        def _():
            acc[...] = jnp.zeros_like(acc)

        acc[...] += jnp.concatenate(
            [jnp.sum(hn, 0, keepdims=True), jnp.sum(hn * hn, 0, keepdims=True)], 0)
        st_ref[...] = acc[...]

    return pl.pallas_call(
        body,
        grid=(NBLK,),
        in_specs=[
            pl.BlockSpec((2, BLK, HD), lambda i: (0, i, 0)),
            pl.BlockSpec((2, BLK, HD), lambda i: (0, i, 0)),
            pl.BlockSpec((2, BLK, 16), lambda i: (0, i, 0)),
            pl.BlockSpec((1, D), lambda i: (0, 0)),
        ],
        out_specs=[
            pl.BlockSpec((BLK, D), lambda i: (i, 0)),
            pl.BlockSpec((2, D), lambda i: (0, 0)),
        ],
        out_shape=[
            jax.ShapeDtypeStruct((N, D), jnp.float32),
            jax.ShapeDtypeStruct((2, D), jnp.float32),
        ],
        scratch_shapes=[pltpu.VMEM((2, D), jnp.float32)],
    )(p, hsplit, degp, b_row)


def _ku_pool(p, hsplit, degp, b_row, batch3):
    def body(p_ref, hs_ref, d_ref, b_ref, seg_ref, g_ref, acc):
        i = pl.program_id(0)
        hn = jnp.maximum(
            _dis(d_ref) * (_hs_full(p_ref) + _hs_full(hs_ref)) + b_ref[...], 0.0)
        seg = seg_ref[0]
        oh = (lax.broadcasted_iota(jnp.int32, (G, BLK), 0) == seg
              ).astype(jnp.float32)

        @pl.when(i == 0)
        def _():
            acc[...] = jnp.zeros_like(acc)

        acc[...] += jnp.dot(oh, hn, preferred_element_type=jnp.float32)
        g_ref[...] = acc[...]

    return pl.pallas_call(
        body,
        grid=(NBLK,),
        in_specs=[
            pl.BlockSpec((2, BLK, HD), lambda i: (0, i, 0)),
            pl.BlockSpec((2, BLK, HD), lambda i: (0, i, 0)),
            pl.BlockSpec((2, BLK, 16), lambda i: (0, i, 0)),
            pl.BlockSpec((1, D), lambda i: (0, 0)),
            pl.BlockSpec((1, 1, BLK), lambda i: (i, 0, 0)),
        ],
        out_specs=pl.BlockSpec((G, D), lambda i: (0, 0)),
        out_shape=jax.ShapeDtypeStruct((G, D), jnp.float32),
        scratch_shapes=[pltpu.VMEM((G, D), jnp.float32)],
    )(p, hsplit, degp, b_row, batch3)


def _head(g, g4, b4, g5, b5, W_fc, bfc, W_cls, bcls):
    def body(g_ref, g4_ref, b4_ref, g5_ref, b5_ref, wfc_ref, bfc_ref,
             wcls_ref, bcls_ref, o_ref):
        gg = g_ref[...]
        m = jnp.mean(gg, 0, keepdims=True)
        v = jnp.mean(gg * gg, 0, keepdims=True) - m * m
        h = g4_ref[...] * lax.rsqrt(v + EPS) * (gg - m) + b4_ref[...]
        h = jnp.maximum(
            jnp.dot(h, wfc_ref[...], preferred_element_type=jnp.float32)
            + bfc_ref[...], 0.0)
        m2 = jnp.mean(h, 0, keepdims=True)
        v2 = jnp.mean(h * h, 0, keepdims=True) - m2 * m2
        h = g5_ref[...] * lax.rsqrt(v2 + EPS) * (h - m2) + b5_ref[...]
        lo = (jnp.dot(h, wcls_ref[...], preferred_element_type=jnp.float32)
              + bcls_ref[...])
        mx = jnp.max(lo, -1, keepdims=True)
        ls = mx + jnp.log(jnp.sum(jnp.exp(lo - mx), -1, keepdims=True))
        o_ref[...] = lo - ls

    return pl.pallas_call(
        body,
        out_shape=jax.ShapeDtypeStruct((G, NCLS), jnp.float32),
    )(g, g4, b4, g5, b5, W_fc, bfc, W_cls, bcls)


def kernel(x, edge_index, batch, W_feat, W1, b1, W2, b2, W3, b3,
           W_fc, b_fc, W_cls, b_cls, bn_g, bn_b):
    row = edge_index[0]
    col = edge_index[1]
    pad = EP - E
    eids = jnp.arange(pad, dtype=jnp.int32)
    row_g = jnp.concatenate([row, eids % 16]).reshape(NSUB, CH2, LW)
    col_s = jnp.concatenate([col, N + (eids % 16)]).reshape(NSUB, CH2, LW)
    row_d = jnp.concatenate([row, N + (eids % 16)]).reshape(NTILES, CH, LW)

    degp = _sc_degree(row_d)
    stats_x = _stats(x)
    h, st = _layer0(x, W_feat, stats_x, bn_g[0:1], bn_b[0:1])
    g = None
    for li, (W, b) in enumerate(((W1, b1), (W2, b2), (W3, b3))):
        hsplit = _kt(h, W, st, bn_g[1 + li:2 + li], bn_b[1 + li:2 + li], degp)
        p = _sc_propagate(hsplit, row_g, col_s)
        if li < 2:
            h, st = _ku(p, hsplit, degp, b.reshape(1, D))
        else:
            g = _ku_pool(p, hsplit, degp, b.reshape(1, D),
                         batch.reshape(NBLK, 1, BLK))
    return _head(g, bn_g[4:5], bn_b[4:5], bn_g[5:6], bn_b[5:6],
                 W_fc, b_fc.reshape(1, D), W_cls, b_cls.reshape(1, NCLS))


# speedup vs baseline: 19.8239x; 1.2573x over previous
import functools

import jax
import jax.numpy as jnp
from jax import lax
from jax.experimental import pallas as pl
from jax.experimental.pallas import tpu as pltpu
from jax.experimental.pallas import tpu_sc as plsc

N = 10000
D = 128
HD = 64
G = 128
NCLS = 10
NP = 10240
E = 320000
NCORES = 2
NSUB = 16
NTILES = NCORES * NSUB
LW = 128
CH = 80
CH2 = 160
EP = NTILES * CH * LW
RPT = NP // NSUB
BLK = 1000
NBLK = 10
EPS = 1e-5

_mesh = plsc.VectorSubcoreMesh(core_axis_name="c", subcore_axis_name="s")


def _sc_degree(row_d):
    @functools.partial(
        pl.kernel,
        out_type=jax.ShapeDtypeStruct((NCORES, NP, 16), jnp.float32),
        mesh=_mesh,
        scratch_types=[
            pltpu.VMEM((CH, LW), jnp.int32),
            pltpu.VMEM((LW, 16), jnp.float32),
            pltpu.VMEM((LW, 16), jnp.float32),
            pltpu.VMEM_SHARED((NP, 16), jnp.float32),
        ],
    )
    def k(row_hbm, out_hbm, row_v, ones_v, zero_v, acc):
        c = lax.axis_index("c")
        s = lax.axis_index("s")
        wid = c * NSUB + s
        pltpu.sync_copy(row_hbm.at[wid], row_v)

        def fill(r, carry):
            ones_v[r, pl.ds(0, 16)] = jnp.ones((16,), jnp.float32)
            zero_v[r, pl.ds(0, 16)] = jnp.zeros((16,), jnp.float32)
            return carry

        lax.fori_loop(0, LW, fill, 0)
        for t in range(RPT // LW):
            pltpu.sync_copy(zero_v, acc.at[pl.ds(s * RPT + t * LW, LW)])
        plsc.subcore_barrier()

        def body(j, carry):
            pltpu.sync_copy(ones_v, acc.at[row_v.at[j]], add=True)
            return carry

        lax.fori_loop(0, CH, body, 0)
        plsc.subcore_barrier()
        pltpu.sync_copy(acc.at[pl.ds(s * RPT, RPT)],
                        out_hbm.at[c, pl.ds(s * RPT, RPT)])

    return k(row_d)


def _sc_propagate(hsplit, row_g, col_s):
    @functools.partial(
        pl.kernel,
        out_type=jax.ShapeDtypeStruct((NCORES, NP, HD), jnp.float32),
        mesh=_mesh,
        scratch_types=[
            pltpu.VMEM((CH2, LW), jnp.int32),
            pltpu.VMEM((CH2, LW), jnp.int32),
            pltpu.VMEM((LW, HD), jnp.float32),
            pltpu.VMEM((LW, HD), jnp.float32),
            pltpu.VMEM((LW, HD), jnp.float32),
            pltpu.VMEM((LW, HD), jnp.float32),
            pltpu.VMEM((LW, HD), jnp.float32),
            pltpu.VMEM_SHARED((NP, HD), jnp.float32),
            pltpu.SemaphoreType.DMA,
            pltpu.SemaphoreType.DMA,
            pltpu.SemaphoreType.DMA,
            pltpu.SemaphoreType.DMA,
            pltpu.SemaphoreType.DMA,
            pltpu.SemaphoreType.DMA,
            pltpu.SemaphoreType.DMA,
            pltpu.SemaphoreType.DMA,
        ],
        compiler_params=pltpu.CompilerParams(use_tc_tiling_on_sc=False),
    )
    def k(hs_hbm, row_hbm, col_hbm, out_hbm, row_v, col_v,
          gbuf0, gbuf1, gbuf2, gbuf3, zbuf, acc,
          gsem0, gsem1, gsem2, gsem3, ssem0, ssem1, ssem2, ssem3):
        c = lax.axis_index("c")
        s = lax.axis_index("s")
        pltpu.sync_copy(row_hbm.at[s], row_v)
        pltpu.sync_copy(col_hbm.at[s], col_v)

        def zrow(r, carry):
            for j in range(HD // 16):
                zbuf[r, pl.ds(16 * j, 16)] = jnp.zeros((16,), jnp.float32)
            return carry

        lax.fori_loop(0, LW, zrow, 0)
        for t in range(RPT // LW):
            pltpu.sync_copy(zbuf, acc.at[pl.ds(s * RPT + t * LW, LW)])
        plsc.subcore_barrier()

        bufs = ((gbuf0, gsem0, ssem0), (gbuf1, gsem1, ssem1),
                (gbuf2, gsem2, ssem2), (gbuf3, gsem3, ssem3))
        pltpu.async_copy(hs_hbm.at[c].at[row_v.at[0]], gbuf0, gsem0)
        pltpu.async_copy(hs_hbm.at[c].at[row_v.at[1]], gbuf1, gsem1)

        def body(i, carry):
            j0 = 4 * i
            for u in range(4):
                gb, gsem, ssem = bufs[u]
                gb2, gsem2_, ssem2_ = bufs[(u + 2) % 4]
                j = j0 + u
                pltpu.make_async_copy(
                    hs_hbm.at[c].at[row_v.at[j]], gb, gsem).wait()
                pltpu.async_copy(gb, acc.at[col_v.at[j]], ssem, add=True)
                if u >= 2:
                    pltpu.make_async_copy(
                        gb2, acc.at[col_v.at[j - 2]], ssem2_).wait()

                    @pl.when(j + 2 < CH2)
                    def _():
                        pltpu.async_copy(
                            hs_hbm.at[c].at[row_v.at[j + 2]], gb2, gsem2_)
                else:
                    @pl.when(i > 0)
                    def _():
                        pltpu.make_async_copy(
                            gb2, acc.at[col_v.at[j - 2]], ssem2_).wait()

                    pltpu.async_copy(
                        hs_hbm.at[c].at[row_v.at[j + 2]], gb2, gsem2_)
            return carry

        lax.fori_loop(0, CH2 // 4, body, 0)
        pltpu.make_async_copy(gbuf2, acc.at[col_v.at[CH2 - 2]], ssem2).wait()
        pltpu.make_async_copy(gbuf3, acc.at[col_v.at[CH2 - 1]], ssem3).wait()
        plsc.subcore_barrier()
        pltpu.sync_copy(acc.at[pl.ds(s * RPT, RPT)],
                        out_hbm.at[c, pl.ds(s * RPT, RPT)])

    return k(hsplit, row_g, col_s)


def _stats(arr):
    def body(x_ref, o_ref, acc):
        i = pl.program_id(0)

        @pl.when(i == 0)
        def _():
            acc[...] = jnp.zeros_like(acc)

        xb = x_ref[...]
        acc[...] += jnp.concatenate(
            [jnp.sum(xb, 0, keepdims=True), jnp.sum(xb * xb, 0, keepdims=True)], 0)
        o_ref[...] = acc[...]

    return pl.pallas_call(
        body,
        grid=(NBLK,),
        in_specs=[pl.BlockSpec((BLK, D), lambda i: (i, 0))],
        out_specs=pl.BlockSpec((2, D), lambda i: (0, 0)),
        out_shape=jax.ShapeDtypeStruct((2, D), jnp.float32),
        scratch_shapes=[pltpu.VMEM((2, D), jnp.float32)],
    )(arr)


def _bn_coeff(s_ref, g_ref, b_ref):
    s = s_ref[...]
    m = s[0:1, :] / N
    v = s[1:2, :] / N - m * m
    a = g_ref[...] * lax.rsqrt(v + EPS)
    return a, b_ref[...] - a * m


def _layer0(x, W_feat, stats_x, g0, b0):
    def body(x_ref, w_ref, s_ref, g_ref, b_ref, h_ref, st_ref, acc):
        i = pl.program_id(0)
        a, cc = _bn_coeff(s_ref, g_ref, b_ref)
        h = jnp.maximum(
            jnp.dot(x_ref[...] * a + cc, w_ref[...],
                    preferred_element_type=jnp.float32), 0.0)
        h_ref[...] = h

        @pl.when(i == 0)
        def _():
            acc[...] = jnp.zeros_like(acc)

        acc[...] += jnp.concatenate(
            [jnp.sum(h, 0, keepdims=True), jnp.sum(h * h, 0, keepdims=True)], 0)
        st_ref[...] = acc[...]

    return pl.pallas_call(
        body,
        grid=(NBLK,),
        in_specs=[
            pl.BlockSpec((BLK, D), lambda i: (i, 0)),
            pl.BlockSpec((D, D), lambda i: (0, 0)),
            pl.BlockSpec((2, D), lambda i: (0, 0)),
            pl.BlockSpec((1, D), lambda i: (0, 0)),
            pl.BlockSpec((1, D), lambda i: (0, 0)),
        ],
        out_specs=[
            pl.BlockSpec((BLK, D), lambda i: (i, 0)),
            pl.BlockSpec((2, D), lambda i: (0, 0)),
        ],
        out_shape=[
            jax.ShapeDtypeStruct((N, D), jnp.float32),
            jax.ShapeDtypeStruct((2, D), jnp.float32),
        ],
        scratch_shapes=[pltpu.VMEM((2, D), jnp.float32)],
    )(x, W_feat, stats_x, g0, b0)


def _dis(d_ref):
    cnt = d_ref[0, :, 0:1] + d_ref[1, :, 0:1] + 1.0
    return lax.rsqrt(cnt)


def _kt(h, W, stats, g_row, b_row, degp):
    def body(h_ref, w_ref, s_ref, g_ref, b_ref, d_ref, hs_ref):
        a, cc = _bn_coeff(s_ref, g_ref, b_ref)
        hb = h_ref[...] * a + cc
        hs = _dis(d_ref) * jnp.dot(
            hb, w_ref[...], preferred_element_type=jnp.float32)
        hs_ref[...] = jnp.stack([hs[:, :HD], hs[:, HD:]], axis=0)

    return pl.pallas_call(
        body,
        grid=(NBLK,),
        in_specs=[
            pl.BlockSpec((BLK, D), lambda i: (i, 0)),
            pl.BlockSpec((D, D), lambda i: (0, 0)),
            pl.BlockSpec((2, D), lambda i: (0, 0)),
            pl.BlockSpec((1, D), lambda i: (0, 0)),
            pl.BlockSpec((1, D), lambda i: (0, 0)),
            pl.BlockSpec((2, BLK, 16), lambda i: (0, i, 0)),
        ],
        out_specs=pl.BlockSpec((2, BLK, HD), lambda i: (0, i, 0)),
        out_shape=jax.ShapeDtypeStruct((2, N, HD), jnp.float32),
    )(h, W, stats, g_row, b_row, degp)


def _hs_full(hs_ref):
    return jnp.concatenate([hs_ref[0], hs_ref[1]], axis=1)


def _ku(p, hsplit, degp, b_row):
    def body(p_ref, hs_ref, d_ref, b_ref, h_ref, st_ref, acc):
        i = pl.program_id(0)
        hn = jnp.maximum(
            _dis(d_ref) * (_hs_full(p_ref) + _hs_full(hs_ref)) + b_ref[...], 0.0)
        h_ref[...] = hn

        @pl.when(i == 0)
        def _():
            acc[...] = jnp.zeros_like(acc)

        acc[...] += jnp.concatenate(
            [jnp.sum(hn, 0, keepdims=True), jnp.sum(hn * hn, 0, keepdims=True)], 0)
        st_ref[...] = acc[...]

    return pl.pallas_call(
        body,
        grid=(NBLK,),
        in_specs=[
            pl.BlockSpec((2, BLK, HD), lambda i: (0, i, 0)),
            pl.BlockSpec((2, BLK, HD), lambda i: (0, i, 0)),
            pl.BlockSpec((2, BLK, 16), lambda i: (0, i, 0)),
            pl.BlockSpec((1, D), lambda i: (0, 0)),
        ],
        out_specs=[
            pl.BlockSpec((BLK, D), lambda i: (i, 0)),
            pl.BlockSpec((2, D), lambda i: (0, 0)),
        ],
        out_shape=[
            jax.ShapeDtypeStruct((N, D), jnp.float32),
            jax.ShapeDtypeStruct((2, D), jnp.float32),
        ],
        scratch_shapes=[pltpu.VMEM((2, D), jnp.float32)],
    )(p, hsplit, degp, b_row)


def _ku_pool(p, hsplit, degp, b_row, batch3):
    def body(p_ref, hs_ref, d_ref, b_ref, seg_ref, g_ref, acc):
        i = pl.program_id(0)
        hn = jnp.maximum(
            _dis(d_ref) * (_hs_full(p_ref) + _hs_full(hs_ref)) + b_ref[...], 0.0)
        seg = seg_ref[0]
        oh = (lax.broadcasted_iota(jnp.int32, (G, BLK), 0) == seg
              ).astype(jnp.float32)

        @pl.when(i == 0)
        def _():
            acc[...] = jnp.zeros_like(acc)

        acc[...] += jnp.dot(oh, hn, preferred_element_type=jnp.float32)
        g_ref[...] = acc[...]

    return pl.pallas_call(
        body,
        grid=(NBLK,),
        in_specs=[
            pl.BlockSpec((2, BLK, HD), lambda i: (0, i, 0)),
            pl.BlockSpec((2, BLK, HD), lambda i: (0, i, 0)),
            pl.BlockSpec((2, BLK, 16), lambda i: (0, i, 0)),
            pl.BlockSpec((1, D), lambda i: (0, 0)),
            pl.BlockSpec((1, 1, BLK), lambda i: (i, 0, 0)),
        ],
        out_specs=pl.BlockSpec((G, D), lambda i: (0, 0)),
        out_shape=jax.ShapeDtypeStruct((G, D), jnp.float32),
        scratch_shapes=[pltpu.VMEM((G, D), jnp.float32)],
    )(p, hsplit, degp, b_row, batch3)


def _head(g, g4, b4, g5, b5, W_fc, bfc, W_cls, bcls):
    def body(g_ref, g4_ref, b4_ref, g5_ref, b5_ref, wfc_ref, bfc_ref,
             wcls_ref, bcls_ref, o_ref):
        gg = g_ref[...]
        m = jnp.mean(gg, 0, keepdims=True)
        v = jnp.mean(gg * gg, 0, keepdims=True) - m * m
        h = g4_ref[...] * lax.rsqrt(v + EPS) * (gg - m) + b4_ref[...]
        h = jnp.maximum(
            jnp.dot(h, wfc_ref[...], preferred_element_type=jnp.float32)
            + bfc_ref[...], 0.0)
        m2 = jnp.mean(h, 0, keepdims=True)
        v2 = jnp.mean(h * h, 0, keepdims=True) - m2 * m2
        h = g5_ref[...] * lax.rsqrt(v2 + EPS) * (h - m2) + b5_ref[...]
        lo = (jnp.dot(h, wcls_ref[...], preferred_element_type=jnp.float32)
              + bcls_ref[...])
        mx = jnp.max(lo, -1, keepdims=True)
        ls = mx + jnp.log(jnp.sum(jnp.exp(lo - mx), -1, keepdims=True))
        o_ref[...] = lo - ls

    return pl.pallas_call(
        body,
        out_shape=jax.ShapeDtypeStruct((G, NCLS), jnp.float32),
    )(g, g4, b4, g5, b5, W_fc, bfc, W_cls, bcls)


def kernel(x, edge_index, batch, W_feat, W1, b1, W2, b2, W3, b3,
           W_fc, b_fc, W_cls, b_cls, bn_g, bn_b):
    row = edge_index[0]
    col = edge_index[1]
    pad = EP - E
    eids = jnp.arange(pad, dtype=jnp.int32)
    row_g = jnp.concatenate([row, eids % 16]).reshape(NSUB, CH2, LW)
    col_s = jnp.concatenate([col, N + (eids % 16)]).reshape(NSUB, CH2, LW)
    row_d = jnp.concatenate([row, N + (eids % 16)]).reshape(NTILES, CH, LW)

    degp = _sc_degree(row_d)
    stats_x = _stats(x)
    h, st = _layer0(x, W_feat, stats_x, bn_g[0:1], bn_b[0:1])
    g = None
    for li, (W, b) in enumerate(((W1, b1), (W2, b2), (W3, b3))):
        hsplit = _kt(h, W, st, bn_g[1 + li:2 + li], bn_b[1 + li:2 + li], degp)
        p = _sc_propagate(hsplit, row_g, col_s)
        if li < 2:
            h, st = _ku(p, hsplit, degp, b.reshape(1, D))
        else:
            g = _ku_pool(p, hsplit, degp, b.reshape(1, D),
                         batch.reshape(NBLK, 1, BLK))
    return _head(g, bn_g[4:5], bn_b[4:5], bn_g[5:6], bn_b[5:6],
                 W_fc, b_fc.reshape(1, D), W_cls, b_cls.reshape(1, NCLS))


# speedup vs baseline: 21.1044x; 1.0646x over previous
import functools

import jax
import jax.numpy as jnp
from jax import lax
from jax.experimental import pallas as pl
from jax.experimental.pallas import tpu as pltpu
from jax.experimental.pallas import tpu_sc as plsc

N = 10000
D = 128
HD = 64
G = 128
NCLS = 10
NP = 10240
E = 320000
NCORES = 2
NSUB = 16
NTILES = NCORES * NSUB
LW = 128
CH = 80
CH2 = 160
EP = NTILES * CH * LW
RPT = NP // NSUB
BLK = 1000
NBLK = 10
EPS = 1e-5

_mesh = plsc.VectorSubcoreMesh(core_axis_name="c", subcore_axis_name="s")


def _sc_degree(row_d):
    @functools.partial(
        pl.kernel,
        out_type=jax.ShapeDtypeStruct((NCORES, NP, 16), jnp.float32),
        mesh=_mesh,
        scratch_types=[
            pltpu.VMEM((CH, LW), jnp.int32),
            pltpu.VMEM((LW, 16), jnp.float32),
            pltpu.VMEM((LW, 16), jnp.float32),
            pltpu.VMEM_SHARED((NP, 16), jnp.float32),
        ],
    )
    def k(row_hbm, out_hbm, row_v, ones_v, zero_v, acc):
        c = lax.axis_index("c")
        s = lax.axis_index("s")
        wid = c * NSUB + s
        pltpu.sync_copy(row_hbm.at[wid], row_v)

        def fill(r, carry):
            ones_v[r, pl.ds(0, 16)] = jnp.ones((16,), jnp.float32)
            zero_v[r, pl.ds(0, 16)] = jnp.zeros((16,), jnp.float32)
            return carry

        lax.fori_loop(0, LW, fill, 0)
        for t in range(RPT // LW):
            pltpu.sync_copy(zero_v, acc.at[pl.ds(s * RPT + t * LW, LW)])
        plsc.subcore_barrier()

        def body(j, carry):
            pltpu.sync_copy(ones_v, acc.at[row_v.at[j]], add=True)
            return carry

        lax.fori_loop(0, CH, body, 0)
        plsc.subcore_barrier()
        pltpu.sync_copy(acc.at[pl.ds(s * RPT, RPT)],
                        out_hbm.at[c, pl.ds(s * RPT, RPT)])

    return k(row_d)


def _sc_propagate(hsplit, row_g, col_s):
    @functools.partial(
        pl.kernel,
        out_type=jax.ShapeDtypeStruct((NCORES, NP, HD), jnp.float32),
        mesh=_mesh,
        scratch_types=[
            pltpu.VMEM((CH2, LW), jnp.int32),
            pltpu.VMEM((CH2, LW), jnp.int32),
            pltpu.VMEM((LW, HD), jnp.float32),
            pltpu.VMEM((LW, HD), jnp.float32),
            pltpu.VMEM((LW, HD), jnp.float32),
            pltpu.VMEM((LW, HD), jnp.float32),
            pltpu.VMEM((LW, HD), jnp.float32),
            pltpu.VMEM_SHARED((NP, HD), jnp.float32),
            pltpu.SemaphoreType.DMA,
            pltpu.SemaphoreType.DMA,
            pltpu.SemaphoreType.DMA,
            pltpu.SemaphoreType.DMA,
            pltpu.SemaphoreType.DMA,
            pltpu.SemaphoreType.DMA,
            pltpu.SemaphoreType.DMA,
            pltpu.SemaphoreType.DMA,
        ],
        compiler_params=pltpu.CompilerParams(use_tc_tiling_on_sc=False),
    )
    def k(hs_hbm, row_hbm, col_hbm, out_hbm, row_v, col_v,
          gbuf0, gbuf1, gbuf2, gbuf3, zbuf, acc,
          gsem0, gsem1, gsem2, gsem3, ssem0, ssem1, ssem2, ssem3):
        c = lax.axis_index("c")
        s = lax.axis_index("s")
        pltpu.sync_copy(row_hbm.at[s], row_v)
        pltpu.sync_copy(col_hbm.at[s], col_v)

        def zrow(r, carry):
            for j in range(HD // 16):
                zbuf[r, pl.ds(16 * j, 16)] = jnp.zeros((16,), jnp.float32)
            return carry

        lax.fori_loop(0, LW, zrow, 0)
        for t in range(RPT // LW):
            pltpu.sync_copy(zbuf, acc.at[pl.ds(s * RPT + t * LW, LW)])
        plsc.subcore_barrier()

        bufs = ((gbuf0, gsem0, ssem0), (gbuf1, gsem1, ssem1),
                (gbuf2, gsem2, ssem2), (gbuf3, gsem3, ssem3))
        pltpu.async_copy(hs_hbm.at[c].at[row_v.at[0]], gbuf0, gsem0)
        pltpu.async_copy(hs_hbm.at[c].at[row_v.at[1]], gbuf1, gsem1)

        def body(i, carry):
            j0 = 4 * i
            for u in range(4):
                gb, gsem, ssem = bufs[u]
                gbp, _, ssemp = bufs[(u + 3) % 4]
                gb2, gsem2_, _ = bufs[(u + 2) % 4]
                j = j0 + u
                pltpu.make_async_copy(
                    hs_hbm.at[c].at[row_v.at[j]], gb, gsem).wait()
                if u == 0:
                    @pl.when(i > 0)
                    def _():
                        pltpu.make_async_copy(
                            gbp, acc.at[col_v.at[j - 1]], ssemp).wait()
                else:
                    pltpu.make_async_copy(
                        gbp, acc.at[col_v.at[j - 1]], ssemp).wait()
                pltpu.async_copy(gb, acc.at[col_v.at[j]], ssem, add=True)
                if u >= 2:
                    @pl.when(j + 2 < CH2)
                    def _():
                        pltpu.async_copy(
                            hs_hbm.at[c].at[row_v.at[j + 2]], gb2, gsem2_)
                else:
                    pltpu.async_copy(
                        hs_hbm.at[c].at[row_v.at[j + 2]], gb2, gsem2_)
            return carry

        lax.fori_loop(0, CH2 // 4, body, 0)
        pltpu.make_async_copy(gbuf3, acc.at[col_v.at[CH2 - 1]], ssem3).wait()
        plsc.subcore_barrier()
        pltpu.sync_copy(acc.at[pl.ds(s * RPT, RPT)],
                        out_hbm.at[c, pl.ds(s * RPT, RPT)])

    return k(hsplit, row_g, col_s)


def _stats(arr):
    def body(x_ref, o_ref, acc):
        i = pl.program_id(0)

        @pl.when(i == 0)
        def _():
            acc[...] = jnp.zeros_like(acc)

        xb = x_ref[...]
        acc[...] += jnp.concatenate(
            [jnp.sum(xb, 0, keepdims=True), jnp.sum(xb * xb, 0, keepdims=True)], 0)
        o_ref[...] = acc[...]

    return pl.pallas_call(
        body,
        grid=(NBLK,),
        in_specs=[pl.BlockSpec((BLK, D), lambda i: (i, 0))],
        out_specs=pl.BlockSpec((2, D), lambda i: (0, 0)),
        out_shape=jax.ShapeDtypeStruct((2, D), jnp.float32),
        scratch_shapes=[pltpu.VMEM((2, D), jnp.float32)],
    )(arr)


def _bn_coeff(s_ref, g_ref, b_ref):
    s = s_ref[...]
    m = s[0:1, :] / N
    v = s[1:2, :] / N - m * m
    a = g_ref[...] * lax.rsqrt(v + EPS)
    return a, b_ref[...] - a * m


def _layer0(x, W_feat, stats_x, g0, b0):
    def body(x_ref, w_ref, s_ref, g_ref, b_ref, h_ref, st_ref, acc):
        i = pl.program_id(0)
        a, cc = _bn_coeff(s_ref, g_ref, b_ref)
        h = jnp.maximum(
            jnp.dot(x_ref[...] * a + cc, w_ref[...],
                    preferred_element_type=jnp.float32), 0.0)
        h_ref[...] = h

        @pl.when(i == 0)
        def _():
            acc[...] = jnp.zeros_like(acc)

        acc[...] += jnp.concatenate(
            [jnp.sum(h, 0, keepdims=True), jnp.sum(h * h, 0, keepdims=True)], 0)
        st_ref[...] = acc[...]

    return pl.pallas_call(
        body,
        grid=(NBLK,),
        in_specs=[
            pl.BlockSpec((BLK, D), lambda i: (i, 0)),
            pl.BlockSpec((D, D), lambda i: (0, 0)),
            pl.BlockSpec((2, D), lambda i: (0, 0)),
            pl.BlockSpec((1, D), lambda i: (0, 0)),
            pl.BlockSpec((1, D), lambda i: (0, 0)),
        ],
        out_specs=[
            pl.BlockSpec((BLK, D), lambda i: (i, 0)),
            pl.BlockSpec((2, D), lambda i: (0, 0)),
        ],
        out_shape=[
            jax.ShapeDtypeStruct((N, D), jnp.float32),
            jax.ShapeDtypeStruct((2, D), jnp.float32),
        ],
        scratch_shapes=[pltpu.VMEM((2, D), jnp.float32)],
    )(x, W_feat, stats_x, g0, b0)


def _dis(d_ref):
    cnt = d_ref[0, :, 0:1] + d_ref[1, :, 0:1] + 1.0
    return lax.rsqrt(cnt)


def _kt(h, W, stats, g_row, b_row, degp):
    def body(h_ref, w_ref, s_ref, g_ref, b_ref, d_ref, hs_ref):
        a, cc = _bn_coeff(s_ref, g_ref, b_ref)
        hb = h_ref[...] * a + cc
        hs = _dis(d_ref) * jnp.dot(
            hb, w_ref[...], preferred_element_type=jnp.float32)
        hs_ref[...] = jnp.stack([hs[:, :HD], hs[:, HD:]], axis=0)

    return pl.pallas_call(
        body,
        grid=(NBLK,),
        in_specs=[
            pl.BlockSpec((BLK, D), lambda i: (i, 0)),
            pl.BlockSpec((D, D), lambda i: (0, 0)),
            pl.BlockSpec((2, D), lambda i: (0, 0)),
            pl.BlockSpec((1, D), lambda i: (0, 0)),
            pl.BlockSpec((1, D), lambda i: (0, 0)),
            pl.BlockSpec((2, BLK, 16), lambda i: (0, i, 0)),
        ],
        out_specs=pl.BlockSpec((2, BLK, HD), lambda i: (0, i, 0)),
        out_shape=jax.ShapeDtypeStruct((2, N, HD), jnp.float32),
    )(h, W, stats, g_row, b_row, degp)


def _hs_full(hs_ref):
    return jnp.concatenate([hs_ref[0], hs_ref[1]], axis=1)


def _ku(p, hsplit, degp, b_row):
    def body(p_ref, hs_ref, d_ref, b_ref, h_ref, st_ref, acc):
        i = pl.program_id(0)
        hn = jnp.maximum(
            _dis(d_ref) * (_hs_full(p_ref) + _hs_full(hs_ref)) + b_ref[...], 0.0)
        h_ref[...] = hn

        @pl.when(i == 0)
        def _():
            acc[...] = jnp.zeros_like(acc)

        acc[...] += jnp.concatenate(
            [jnp.sum(hn, 0, keepdims=True), jnp.sum(hn * hn, 0, keepdims=True)], 0)
        st_ref[...] = acc[...]

    return pl.pallas_call(
        body,
        grid=(NBLK,),
        in_specs=[
            pl.BlockSpec((2, BLK, HD), lambda i: (0, i, 0)),
            pl.BlockSpec((2, BLK, HD), lambda i: (0, i, 0)),
            pl.BlockSpec((2, BLK, 16), lambda i: (0, i, 0)),
            pl.BlockSpec((1, D), lambda i: (0, 0)),
        ],
        out_specs=[
            pl.BlockSpec((BLK, D), lambda i: (i, 0)),
            pl.BlockSpec((2, D), lambda i: (0, 0)),
        ],
        out_shape=[
            jax.ShapeDtypeStruct((N, D), jnp.float32),
            jax.ShapeDtypeStruct((2, D), jnp.float32),
        ],
        scratch_shapes=[pltpu.VMEM((2, D), jnp.float32)],
    )(p, hsplit, degp, b_row)


def _ku_pool(p, hsplit, degp, b_row, batch3):
    def body(p_ref, hs_ref, d_ref, b_ref, seg_ref, g_ref, acc):
        i = pl.program_id(0)
        hn = jnp.maximum(
            _dis(d_ref) * (_hs_full(p_ref) + _hs_full(hs_ref)) + b_ref[...], 0.0)
        seg = seg_ref[0]
        oh = (lax.broadcasted_iota(jnp.int32, (G, BLK), 0) == seg
              ).astype(jnp.float32)

        @pl.when(i == 0)
        def _():
            acc[...] = jnp.zeros_like(acc)

        acc[...] += jnp.dot(oh, hn, preferred_element_type=jnp.float32)
        g_ref[...] = acc[...]

    return pl.pallas_call(
        body,
        grid=(NBLK,),
        in_specs=[
            pl.BlockSpec((2, BLK, HD), lambda i: (0, i, 0)),
            pl.BlockSpec((2, BLK, HD), lambda i: (0, i, 0)),
            pl.BlockSpec((2, BLK, 16), lambda i: (0, i, 0)),
            pl.BlockSpec((1, D), lambda i: (0, 0)),
            pl.BlockSpec((1, 1, BLK), lambda i: (i, 0, 0)),
        ],
        out_specs=pl.BlockSpec((G, D), lambda i: (0, 0)),
        out_shape=jax.ShapeDtypeStruct((G, D), jnp.float32),
        scratch_shapes=[pltpu.VMEM((G, D), jnp.float32)],
    )(p, hsplit, degp, b_row, batch3)


def _head(g, g4, b4, g5, b5, W_fc, bfc, W_cls, bcls):
    def body(g_ref, g4_ref, b4_ref, g5_ref, b5_ref, wfc_ref, bfc_ref,
             wcls_ref, bcls_ref, o_ref):
        gg = g_ref[...]
        m = jnp.mean(gg, 0, keepdims=True)
        v = jnp.mean(gg * gg, 0, keepdims=True) - m * m
        h = g4_ref[...] * lax.rsqrt(v + EPS) * (gg - m) + b4_ref[...]
        h = jnp.maximum(
            jnp.dot(h, wfc_ref[...], preferred_element_type=jnp.float32)
            + bfc_ref[...], 0.0)
        m2 = jnp.mean(h, 0, keepdims=True)
        v2 = jnp.mean(h * h, 0, keepdims=True) - m2 * m2
        h = g5_ref[...] * lax.rsqrt(v2 + EPS) * (h - m2) + b5_ref[...]
        lo = (jnp.dot(h, wcls_ref[...], preferred_element_type=jnp.float32)
              + bcls_ref[...])
        mx = jnp.max(lo, -1, keepdims=True)
        ls = mx + jnp.log(jnp.sum(jnp.exp(lo - mx), -1, keepdims=True))
        o_ref[...] = lo - ls

    return pl.pallas_call(
        body,
        out_shape=jax.ShapeDtypeStruct((G, NCLS), jnp.float32),
    )(g, g4, b4, g5, b5, W_fc, bfc, W_cls, bcls)


def kernel(x, edge_index, batch, W_feat, W1, b1, W2, b2, W3, b3,
           W_fc, b_fc, W_cls, b_cls, bn_g, bn_b):
    row = edge_index[0]
    col = edge_index[1]
    pad = EP - E
    eids = jnp.arange(pad, dtype=jnp.int32)
    row_g = jnp.concatenate([row, eids % 16]).reshape(NSUB, CH2, LW)
    col_s = jnp.concatenate([col, N + (eids % 16)]).reshape(NSUB, CH2, LW)
    row_d = jnp.concatenate([row, N + (eids % 16)]).reshape(NTILES, CH, LW)

    degp = _sc_degree(row_d)
    stats_x = _stats(x)
    h, st = _layer0(x, W_feat, stats_x, bn_g[0:1], bn_b[0:1])
    g = None
    for li, (W, b) in enumerate(((W1, b1), (W2, b2), (W3, b3))):
        hsplit = _kt(h, W, st, bn_g[1 + li:2 + li], bn_b[1 + li:2 + li], degp)
        p = _sc_propagate(hsplit, row_g, col_s)
        if li < 2:
            h, st = _ku(p, hsplit, degp, b.reshape(1, D))
        else:
            g = _ku_pool(p, hsplit, degp, b.reshape(1, D),
                         batch.reshape(NBLK, 1, BLK))
    return _head(g, bn_g[4:5], bn_b[4:5], bn_g[5:6], bn_b[5:6],
                 W_fc, b_fc.reshape(1, D), W_cls, b_cls.reshape(1, NCLS))
